# Initial kernel scaffold; baseline (speedup 1.0000x reference)
import functools

import jax
import jax.numpy as jnp
from jax import lax
from jax.experimental import pallas as pl
from jax.experimental.pallas import tpu as pltpu
from jax.experimental.pallas import tpu_sc as plsc

N_NODES = 10000
N_EDGES = 320000
IN_CH = 128
HID = 64
OUT_CH = 128

NC = 2
NS = 16
NW = NC * NS

CHUNK = 128
EPT = 10240
E_PAD = EPT * NW
N_PAD = 10016
ROWS_PER_TILE = N_PAD // NS


def _sc_mesh():
  return plsc.VectorSubcoreMesh(core_axis_name="c", subcore_axis_name="s")


def _deg_body(dst_hbm, ones_hbm, zrows_hbm, out_hbm,
              idx_v, ones_v, zrows_v, table_sh, sem):
  c = lax.axis_index("c")
  s = lax.axis_index("s")
  wid = s * NC + c

  pltpu.sync_copy(zrows_hbm, zrows_v)
  pltpu.sync_copy(zrows_v, table_sh.at[pl.ds(s * ROWS_PER_TILE, ROWS_PER_TILE)])
  pltpu.sync_copy(ones_hbm, ones_v)
  plsc.subcore_barrier()

  base = wid * EPT
  nchunks = EPT // CHUNK

  def step(k, carry):
    pltpu.sync_copy(dst_hbm.at[pl.ds(base + k * CHUNK, CHUNK)], idx_v)
    pltpu.sync_copy(ones_v, table_sh.at[idx_v], add=True)
    return carry

  lax.fori_loop(0, nchunks, step, 0, unroll=False)
  plsc.subcore_barrier()

  row0 = s * ROWS_PER_TILE
  pltpu.async_copy(
      table_sh.at[pl.ds(row0, ROWS_PER_TILE)],
      out_hbm.at[c, pl.ds(row0, ROWS_PER_TILE)],
      sem,
  ).wait()


def _deg_pass(dst_pad, ones, zrows):
  kfn = pl.kernel(
      _deg_body,
      out_type=jax.ShapeDtypeStruct((NC, N_PAD, 16), jnp.float32),
      mesh=_sc_mesh(),
      scratch_types=[
          pltpu.VMEM((CHUNK,), jnp.int32),
          pltpu.VMEM((CHUNK, 16), jnp.float32),
          pltpu.VMEM((ROWS_PER_TILE, 16), jnp.float32),
          pltpu.VMEM_SHARED((N_PAD, 16), jnp.float32),
          pltpu.SemaphoreType.DMA,
      ],
  )
  return kfn(dst_pad, ones, zrows)


def _edge_body(D, ys_hbm, src_hbm, dst_hbm, zrows_hbm, out_hbm,
               src_v, dst_v, rows_v, zrows_v, agg_sh, gsem, ssem, osem):
  c = lax.axis_index("c")
  s = lax.axis_index("s")
  wid = s * NC + c

  pltpu.sync_copy(zrows_hbm, zrows_v)
  pltpu.sync_copy(zrows_v, agg_sh.at[pl.ds(s * ROWS_PER_TILE, ROWS_PER_TILE)])
  plsc.subcore_barrier()

  base = wid * EPT
  nchunks = EPT // CHUNK

  pltpu.sync_copy(src_hbm.at[pl.ds(base, CHUNK)], src_v.at[0])
  pltpu.sync_copy(dst_hbm.at[pl.ds(base, CHUNK)], dst_v.at[0])
  pltpu.async_copy(ys_hbm.at[src_v.at[0]], rows_v.at[0], gsem)

  def step(k, carry):
    cur = lax.rem(k, 2)
    nxt = lax.rem(k + 1, 2)

    @pl.when(k + 1 < nchunks)
    def _prefetch():
      off = base + (k + 1) * CHUNK
      pltpu.sync_copy(src_hbm.at[pl.ds(off, CHUNK)], src_v.at[nxt])
      pltpu.sync_copy(dst_hbm.at[pl.ds(off, CHUNK)], dst_v.at[nxt])
      pltpu.async_copy(ys_hbm.at[src_v.at[nxt]], rows_v.at[nxt], gsem)

    pltpu.make_async_copy(ys_hbm.at[src_v.at[cur]], rows_v.at[cur], gsem).wait()
    pltpu.async_copy(rows_v.at[cur], agg_sh.at[dst_v.at[cur]], ssem,
                     add=True).wait()
    return carry

  lax.fori_loop(0, nchunks, step, 0, unroll=False)
  plsc.subcore_barrier()

  row0 = s * ROWS_PER_TILE
  pltpu.async_copy(
      agg_sh.at[pl.ds(row0, ROWS_PER_TILE)],
      out_hbm.at[c, pl.ds(row0, ROWS_PER_TILE)],
      osem,
  ).wait()


def _edge_pass(ys_pad, src_pad, dst_pad, zrows):
  D = ys_pad.shape[1]
  kfn = pl.kernel(
      functools.partial(_edge_body, D),
      out_type=jax.ShapeDtypeStruct((NC, N_PAD, D), jnp.float32),
      mesh=_sc_mesh(),
      scratch_types=[
          pltpu.VMEM((2, CHUNK), jnp.int32),
          pltpu.VMEM((2, CHUNK), jnp.int32),
          pltpu.VMEM((2, CHUNK, D), jnp.float32),
          pltpu.VMEM((ROWS_PER_TILE, D), jnp.float32),
          pltpu.VMEM_SHARED((N_PAD, D), jnp.float32),
          pltpu.SemaphoreType.DMA,
          pltpu.SemaphoreType.DMA,
          pltpu.SemaphoreType.DMA,
      ],
  )
  return kfn(ys_pad, src_pad, dst_pad, zrows)


def _tc_scale_matmul_body(degp_ref, x_ref, w_ref, ys_ref, dinv_ref):
  deg = degp_ref[0] + degp_ref[1] + 1.0
  dinv = lax.rsqrt(deg)
  dinv_ref[...] = dinv
  xw = jnp.dot(x_ref[...], w_ref[...], preferred_element_type=jnp.float32)
  ys_ref[...] = xw * dinv


def _tc_scale_matmul(degp, x_pad, w):
  d_out = w.shape[1]
  deg_cols = degp[:, :, 0:1]
  return pl.pallas_call(
      _tc_scale_matmul_body,
      out_shape=(
          jax.ShapeDtypeStruct((N_PAD, d_out), jnp.float32),
          jax.ShapeDtypeStruct((N_PAD, 1), jnp.float32),
      ),
  )(deg_cols, x_pad, w)


def _tc_layer1_finish_body(p_ref, ys_ref, dinv_ref, b_ref, w2_ref, ys2_ref):
  agg = p_ref[0] + p_ref[1] + ys_ref[...]
  h = jnp.maximum(agg * dinv_ref[...] + b_ref[...], 0.0)
  hw = jnp.dot(h, w2_ref[...], preferred_element_type=jnp.float32)
  ys2_ref[...] = hw * dinv_ref[...]


def _tc_layer1_finish(partials, ys1, dinv, b1, w2):
  return pl.pallas_call(
      _tc_layer1_finish_body,
      out_shape=jax.ShapeDtypeStruct((N_PAD, OUT_CH), jnp.float32),
  )(partials, ys1, dinv, b1.reshape(1, HID), w2)


def _tc_layer2_finish_body(p_ref, ys_ref, dinv_ref, b_ref, out_ref):
  agg = p_ref[0] + p_ref[1] + ys_ref[...]
  out_ref[...] = agg * dinv_ref[...] + b_ref[...]


def _tc_layer2_finish(partials, ys2, dinv, b2):
  return pl.pallas_call(
      _tc_layer2_finish_body,
      out_shape=jax.ShapeDtypeStruct((N_PAD, OUT_CH), jnp.float32),
  )(partials, ys2, dinv, b2.reshape(1, OUT_CH))


@jax.jit
def _gcn_block(x, edge_index, W1, b1, W2, b2):
  src = edge_index[0].astype(jnp.int32)
  dst = edge_index[1].astype(jnp.int32)
  pad_idx = jnp.full((E_PAD - N_EDGES,), N_NODES, dtype=jnp.int32)
  src_pad = jnp.concatenate([src, pad_idx])
  dst_pad = jnp.concatenate([dst, pad_idx])

  x_pad = jnp.zeros((N_PAD, IN_CH), x.dtype).at[:N_NODES].set(x)

  ones = jnp.ones((CHUNK, 16), jnp.float32)
  zrows16 = jnp.zeros((ROWS_PER_TILE, 16), jnp.float32)
  zrows_h = jnp.zeros((ROWS_PER_TILE, HID), jnp.float32)
  zrows_o = jnp.zeros((ROWS_PER_TILE, OUT_CH), jnp.float32)

  degp = _deg_pass(dst_pad, ones, zrows16)

  ys1, dinv = _tc_scale_matmul(degp, x_pad, W1)
  p1 = _edge_pass(ys1, src_pad, dst_pad, zrows_h)

  ys2 = _tc_layer1_finish(p1, ys1, dinv, b1, W2)
  p2 = _edge_pass(ys2, src_pad, dst_pad, zrows_o)

  out = _tc_layer2_finish(p2, ys2, dinv, b2)
  return out[:N_NODES]


def kernel(x, edge_index, W1, b1, W2, b2):
  return _gcn_block(x, edge_index, W1, b1, W2, b2)


# speedup vs baseline: 11.3651x; 11.3651x over previous
import jax
import jax.numpy as jnp
from jax import lax
from jax.experimental import pallas as pl
from jax.experimental.pallas import tpu as pltpu
from jax.experimental.pallas import tpu_sc as plsc

N_NODES = 10000
N_EDGES = 320000
IN_CH = 128
HID = 64
OUT_CH = 128

NC = 2
NS = 16
NW = NC * NS

CHUNK = 128
EPT = 10240
E_PAD = EPT * NW
N_PAD = 10112
ROWS_PER_TILE = N_PAD // NS
D = 64


def _sc_mesh():
  return plsc.VectorSubcoreMesh(core_axis_name="c", subcore_axis_name="s")


def _deg_body(dst_hbm, ones_hbm, zrows_hbm, out_hbm,
              idx_v, ones_v, zrows_v, table_sh, sem):
  c = lax.axis_index("c")
  s = lax.axis_index("s")
  wid = s * NC + c

  if True:
    pltpu.sync_copy(zrows_hbm, zrows_v)
    pltpu.sync_copy(zrows_v,
                    table_sh.at[pl.ds(s * ROWS_PER_TILE, ROWS_PER_TILE)])
    pltpu.sync_copy(ones_hbm, ones_v)
    plsc.subcore_barrier()

    base = wid * EPT
    nchunks = EPT // CHUNK

    @pl.loop(0, nchunks)
    def _step(k):
      pltpu.sync_copy(dst_hbm.at[pl.ds(base + k * CHUNK, CHUNK)], idx_v)
      pltpu.sync_copy(ones_v, table_sh.at[idx_v], add=True)

    plsc.subcore_barrier()

    row0 = s * ROWS_PER_TILE
    pltpu.async_copy(
        table_sh.at[pl.ds(row0, ROWS_PER_TILE)],
        out_hbm.at[c, pl.ds(row0, ROWS_PER_TILE)],
        sem,
    ).wait()


def _deg_pass(dst_pad, ones, zrows):
  kfn = pl.kernel(
      _deg_body,
      out_type=jax.ShapeDtypeStruct((NC, N_PAD, 16), jnp.float32),
      mesh=_sc_mesh(),
      scratch_types=[
          pltpu.VMEM((CHUNK,), jnp.int32),
          pltpu.VMEM((CHUNK, 16), jnp.float32),
          pltpu.VMEM((ROWS_PER_TILE, 16), jnp.float32),
          pltpu.VMEM_SHARED((N_PAD, 16), jnp.float32),
          pltpu.SemaphoreType.DMA,
      ],
      compiler_params=pltpu.CompilerParams(use_tc_tiling_on_sc=False),
  )
  return kfn(dst_pad, ones, zrows)


def _edge_body(ys_hbm, src_hbm, dst_hbm, zrows_hbm, out_hbm,
               src_v, dst_v, rows_v, zrows_v, agg_sh, gsem, ssem, osem):
  c = lax.axis_index("c")
  s = lax.axis_index("s")
  wid = s * NC + c

  if True:
    pltpu.sync_copy(zrows_hbm, zrows_v)
    pltpu.sync_copy(zrows_v, agg_sh.at[pl.ds(s * ROWS_PER_TILE, ROWS_PER_TILE)])
    plsc.subcore_barrier()

    base = wid * EPT
    nchunks = EPT // CHUNK

    def fetch(k, b):
      off = base + k * CHUNK
      pltpu.sync_copy(src_hbm.at[pl.ds(off, CHUNK)], src_v.at[b])
      pltpu.sync_copy(dst_hbm.at[pl.ds(off, CHUNK)], dst_v.at[b])
      pltpu.async_copy(ys_hbm.at[src_v.at[b]], rows_v.at[b], gsem)

    def drain(b):
      pltpu.make_async_copy(ys_hbm.at[src_v.at[b]], rows_v.at[b], gsem).wait()
      pltpu.async_copy(rows_v.at[b], agg_sh.at[dst_v.at[b]], ssem,
                       add=True).wait()

    fetch(0, 0)

    @pl.loop(0, nchunks, step=2)
    def _chunks(k0):
      fetch(k0 + 1, 1)
      drain(0)

      @pl.when(k0 + 2 < nchunks)
      def _pre():
        fetch(k0 + 2, 0)

      drain(1)

    plsc.subcore_barrier()

    row0 = s * ROWS_PER_TILE
    pltpu.async_copy(
        agg_sh.at[pl.ds(row0, ROWS_PER_TILE)],
        out_hbm.at[c, pl.ds(row0, ROWS_PER_TILE)],
        osem,
    ).wait()


def _edge_pass(ys_pad, src_pad, dst_pad, zrows):
  kfn = pl.kernel(
      _edge_body,
      out_type=jax.ShapeDtypeStruct((NC, N_PAD, D), jnp.float32),
      mesh=_sc_mesh(),
      scratch_types=[
          pltpu.VMEM((2, CHUNK), jnp.int32),
          pltpu.VMEM((2, CHUNK), jnp.int32),
          pltpu.VMEM((2, CHUNK, D), jnp.float32),
          pltpu.VMEM((ROWS_PER_TILE, D), jnp.float32),
          pltpu.VMEM_SHARED((N_PAD, D), jnp.float32),
          pltpu.SemaphoreType.DMA,
          pltpu.SemaphoreType.DMA,
          pltpu.SemaphoreType.DMA,
      ],
      compiler_params=pltpu.CompilerParams(use_tc_tiling_on_sc=False),
  )
  return kfn(ys_pad, src_pad, dst_pad, zrows)


def _tc_scale_matmul_body(degp_ref, x_ref, w_ref, ys_ref, dinv_ref):
  deg = degp_ref[0] + degp_ref[1] + 1.0
  dinv = lax.rsqrt(deg)
  dinv_ref[...] = dinv
  xw = jnp.dot(x_ref[...], w_ref[...], preferred_element_type=jnp.float32)
  ys_ref[...] = xw * dinv


def _tc_scale_matmul(degp, x_pad, w):
  d_out = w.shape[1]
  deg_cols = degp[:, :, 0:1]
  return pl.pallas_call(
      _tc_scale_matmul_body,
      out_shape=(
          jax.ShapeDtypeStruct((N_PAD, d_out), jnp.float32),
          jax.ShapeDtypeStruct((N_PAD, 1), jnp.float32),
      ),
  )(deg_cols, x_pad, w)


def _tc_layer1_finish_body(p_ref, ys_ref, dinv_ref, b_ref, w2_ref,
                           ys2a_ref, ys2b_ref):
  agg = p_ref[0] + p_ref[1] + ys_ref[...]
  h = jnp.maximum(agg * dinv_ref[...] + b_ref[...], 0.0)
  hw = jnp.dot(h, w2_ref[...], preferred_element_type=jnp.float32)
  ys2 = hw * dinv_ref[...]
  ys2a_ref[...] = ys2[:, :D]
  ys2b_ref[...] = ys2[:, D:]


def _tc_layer1_finish(partials, ys1, dinv, b1, w2):
  return pl.pallas_call(
      _tc_layer1_finish_body,
      out_shape=(
          jax.ShapeDtypeStruct((N_PAD, D), jnp.float32),
          jax.ShapeDtypeStruct((N_PAD, D), jnp.float32),
      ),
  )(partials, ys1, dinv, b1.reshape(1, HID), w2)


def _tc_layer2_finish_body(pa_ref, pb_ref, ysa_ref, ysb_ref, dinv_ref, b_ref,
                           out_ref):
  agga = pa_ref[0] + pa_ref[1] + ysa_ref[...]
  aggb = pb_ref[0] + pb_ref[1] + ysb_ref[...]
  out_ref[:, :D] = agga * dinv_ref[...] + b_ref[:, :D]
  out_ref[:, D:] = aggb * dinv_ref[...] + b_ref[:, D:]


def _tc_layer2_finish(p2a, p2b, ys2a, ys2b, dinv, b2):
  return pl.pallas_call(
      _tc_layer2_finish_body,
      out_shape=jax.ShapeDtypeStruct((N_PAD, OUT_CH), jnp.float32),
  )(p2a, p2b, ys2a, ys2b, dinv, b2.reshape(1, OUT_CH))


def _gcn_block(x, edge_index, W1, b1, W2, b2):
  src = edge_index[0].astype(jnp.int32)
  dst = edge_index[1].astype(jnp.int32)
  pad_idx = jnp.full((E_PAD - N_EDGES,), N_NODES, dtype=jnp.int32)
  src_pad = jnp.concatenate([src, pad_idx])
  dst_pad = jnp.concatenate([dst, pad_idx])

  x_pad = jnp.zeros((N_PAD, IN_CH), x.dtype).at[:N_NODES].set(x)

  ones = jnp.ones((CHUNK, 16), jnp.float32)
  zrows16 = jnp.zeros((ROWS_PER_TILE, 16), jnp.float32)
  zrows_d = jnp.zeros((ROWS_PER_TILE, D), jnp.float32)

  degp = _deg_pass(dst_pad, ones, zrows16)

  ys1, dinv = _tc_scale_matmul(degp, x_pad, W1)
  p1 = _edge_pass(ys1, src_pad, dst_pad, zrows_d)

  ys2a, ys2b = _tc_layer1_finish(p1, ys1, dinv, b1, W2)
  p2a = _edge_pass(ys2a, src_pad, dst_pad, zrows_d)
  p2b = _edge_pass(ys2b, src_pad, dst_pad, zrows_d)

  out = _tc_layer2_finish(p2a, p2b, ys2a, ys2b, dinv, b2)
  return out[:N_NODES]


def kernel(x, edge_index, W1, b1, W2, b2):
  return _gcn_block(x, edge_index, W1, b1, W2, b2)


# speedup vs baseline: 32.7523x; 2.8818x over previous
import jax
import jax.numpy as jnp
from jax import lax
from jax.experimental import pallas as pl
from jax.experimental.pallas import tpu as pltpu
from jax.experimental.pallas import tpu_sc as plsc

N_NODES = 10000
N_EDGES = 320000
IN_CH = 128
HID = 64
OUT_CH = 128

NC = 2
NS = 16
NW = NC * NS

CHUNK = 128
EPT = 10240
E_PAD = EPT * NW
NCHUNKS = EPT // CHUNK
N_PAD = 10112
ROWS_PER_TILE = N_PAD // NS
D = 64
NBUF = 4


def _sc_mesh():
  return plsc.VectorSubcoreMesh(core_axis_name="c", subcore_axis_name="s")


def _deg_body(dst3_hbm, ones_hbm, zrows_hbm, out_hbm,
              idx_v, ones_v, zrows_v, table_sh, ssem, osem):
  c = lax.axis_index("c")
  s = lax.axis_index("s")
  wid = s * NC + c

  pltpu.sync_copy(zrows_hbm, zrows_v)
  pltpu.sync_copy(zrows_v, table_sh.at[pl.ds(s * ROWS_PER_TILE, ROWS_PER_TILE)])
  pltpu.sync_copy(ones_hbm, ones_v)
  pltpu.sync_copy(dst3_hbm.at[wid], idx_v)
  plsc.subcore_barrier()

  @pl.loop(0, NCHUNKS)
  def _fire(k):
    pltpu.async_copy(ones_v, table_sh.at[idx_v.at[k]], ssem, add=True)

  @pl.loop(0, NCHUNKS)
  def _drain(k):
    pltpu.make_async_copy(ones_v, table_sh.at[idx_v.at[k]], ssem).wait()

  plsc.subcore_barrier()

  row0 = s * ROWS_PER_TILE
  pltpu.async_copy(
      table_sh.at[pl.ds(row0, ROWS_PER_TILE)],
      out_hbm.at[c, pl.ds(row0, ROWS_PER_TILE)],
      osem,
  ).wait()


def _deg_pass(dst3, ones, zrows):
  kfn = pl.kernel(
      _deg_body,
      out_type=jax.ShapeDtypeStruct((NC, N_PAD, 8), jnp.float32),
      mesh=_sc_mesh(),
      scratch_types=[
          pltpu.VMEM((NCHUNKS, CHUNK), jnp.int32),
          pltpu.VMEM((CHUNK, 8), jnp.float32),
          pltpu.VMEM((ROWS_PER_TILE, 8), jnp.float32),
          pltpu.VMEM_SHARED((N_PAD, 8), jnp.float32),
          pltpu.SemaphoreType.DMA,
          pltpu.SemaphoreType.DMA,
      ],
      compiler_params=pltpu.CompilerParams(use_tc_tiling_on_sc=False),
  )
  return kfn(dst3, ones, zrows)


def _edge_body(ys_hbm, src3_hbm, dst3_hbm, zrows_hbm, out_hbm,
               src_v, dst_v, rows_v, zrows_v, table_sh,
               g0, g1, g2, g3, s0, s1, s2, s3, osem):
  c = lax.axis_index("c")
  s = lax.axis_index("s")
  wid = s * NC + c
  gsems = (g0, g1, g2, g3)
  ssems = (s0, s1, s2, s3)

  pltpu.sync_copy(zrows_hbm, zrows_v)
  row0 = s * ROWS_PER_TILE
  for z0 in (0, 128, 256, 384):
    pltpu.sync_copy(zrows_v, table_sh.at[pl.ds(row0 + z0, 128)])
  pltpu.sync_copy(zrows_v.at[pl.ds(0, ROWS_PER_TILE - 512)],
                  table_sh.at[pl.ds(row0 + 512, ROWS_PER_TILE - 512)])
  pltpu.sync_copy(src3_hbm.at[wid], src_v)
  pltpu.sync_copy(dst3_hbm.at[wid], dst_v)
  plsc.subcore_barrier()

  def gather(k, b):
    pltpu.async_copy(ys_hbm.at[src_v.at[k]], rows_v.at[b], gsems[b])

  def wait_gather(k, b):
    pltpu.make_async_copy(ys_hbm.at[src_v.at[k]], rows_v.at[b], gsems[b]).wait()

  def scatter(k, b):
    pltpu.async_copy(rows_v.at[b], table_sh.at[dst_v.at[k]], ssems[b],
                     add=True)

  def wait_scatter(k, b):
    pltpu.make_async_copy(rows_v.at[b], table_sh.at[dst_v.at[k]],
                          ssems[b]).wait()

  gather(0, 0)
  gather(1, 1)

  @pl.loop(0, NCHUNKS, step=NBUF)
  def _chunks(k0):
    for b in range(NBUF):
      k = k0 + b
      bp2 = (b + 2) % NBUF

      @pl.when(k >= 2)
      def _wait_prev_scatter():
        wait_scatter(k - 2, bp2)

      @pl.when(k + 2 < NCHUNKS)
      def _prefetch():
        gather(k + 2, bp2)

      wait_gather(k, b)
      scatter(k, b)

  wait_scatter(NCHUNKS - 2, (NCHUNKS - 2) % NBUF)
  wait_scatter(NCHUNKS - 1, (NCHUNKS - 1) % NBUF)
  plsc.subcore_barrier()

  pltpu.async_copy(
      table_sh.at[pl.ds(row0, ROWS_PER_TILE)],
      out_hbm.at[c, pl.ds(row0, ROWS_PER_TILE)],
      osem,
  ).wait()


def _edge_pass(ys_pad, src3, dst3, zrows):
  kfn = pl.kernel(
      _edge_body,
      out_type=jax.ShapeDtypeStruct((NC, N_PAD, D), jnp.float32),
      mesh=_sc_mesh(),
      scratch_types=[
          pltpu.VMEM((NCHUNKS, CHUNK), jnp.int32),
          pltpu.VMEM((NCHUNKS, CHUNK), jnp.int32),
          pltpu.VMEM((NBUF, CHUNK, D), jnp.float32),
          pltpu.VMEM((128, D), jnp.float32),
          pltpu.VMEM_SHARED((N_PAD, D), jnp.float32),
          pltpu.SemaphoreType.DMA,
          pltpu.SemaphoreType.DMA,
          pltpu.SemaphoreType.DMA,
          pltpu.SemaphoreType.DMA,
          pltpu.SemaphoreType.DMA,
          pltpu.SemaphoreType.DMA,
          pltpu.SemaphoreType.DMA,
          pltpu.SemaphoreType.DMA,
          pltpu.SemaphoreType.DMA,
      ],
      compiler_params=pltpu.CompilerParams(use_tc_tiling_on_sc=False),
  )
  return kfn(ys_pad, src3, dst3, zrows)


def _tc_scale_matmul_body(degp_ref, x_ref, w_ref, ys_ref, dinv_ref):
  deg = degp_ref[0] + degp_ref[1] + 1.0
  dinv = lax.rsqrt(deg)
  dinv_ref[...] = dinv
  xw = jnp.dot(x_ref[...], w_ref[...], preferred_element_type=jnp.float32)
  ys_ref[...] = xw * dinv


def _tc_scale_matmul(degp, x_pad, w):
  d_out = w.shape[1]
  deg_cols = degp[:, :, 0:1]
  return pl.pallas_call(
      _tc_scale_matmul_body,
      out_shape=(
          jax.ShapeDtypeStruct((N_PAD, d_out), jnp.float32),
          jax.ShapeDtypeStruct((N_PAD, 1), jnp.float32),
      ),
  )(deg_cols, x_pad, w)


def _tc_layer1_finish_body(p_ref, ys_ref, dinv_ref, b_ref, w2_ref,
                           ys2a_ref, ys2b_ref):
  agg = p_ref[0] + p_ref[1] + ys_ref[...]
  h = jnp.maximum(agg * dinv_ref[...] + b_ref[...], 0.0)
  hw = jnp.dot(h, w2_ref[...], preferred_element_type=jnp.float32)
  ys2 = hw * dinv_ref[...]
  ys2a_ref[...] = ys2[:, :D]
  ys2b_ref[...] = ys2[:, D:]


def _tc_layer1_finish(partials, ys1, dinv, b1, w2):
  return pl.pallas_call(
      _tc_layer1_finish_body,
      out_shape=(
          jax.ShapeDtypeStruct((N_PAD, D), jnp.float32),
          jax.ShapeDtypeStruct((N_PAD, D), jnp.float32),
      ),
  )(partials, ys1, dinv, b1.reshape(1, HID), w2)


def _tc_layer2_finish_body(pa_ref, pb_ref, ysa_ref, ysb_ref, dinv_ref, b_ref,
                           out_ref):
  agga = pa_ref[0] + pa_ref[1] + ysa_ref[...]
  aggb = pb_ref[0] + pb_ref[1] + ysb_ref[...]
  out_ref[:, :D] = agga * dinv_ref[...] + b_ref[:, :D]
  out_ref[:, D:] = aggb * dinv_ref[...] + b_ref[:, D:]


def _tc_layer2_finish(p2a, p2b, ys2a, ys2b, dinv, b2):
  return pl.pallas_call(
      _tc_layer2_finish_body,
      out_shape=jax.ShapeDtypeStruct((N_PAD, OUT_CH), jnp.float32),
  )(p2a, p2b, ys2a, ys2b, dinv, b2.reshape(1, OUT_CH))


def _gcn_block(x, edge_index, W1, b1, W2, b2):
  src = edge_index[0].astype(jnp.int32)
  dst = edge_index[1].astype(jnp.int32)
  npad_e = E_PAD - N_EDGES
  pad_idx = N_NODES + (jnp.arange(npad_e, dtype=jnp.int32) % (N_PAD - N_NODES))
  src3 = jnp.concatenate([src, pad_idx]).reshape(NW, NCHUNKS, CHUNK)
  dst3 = jnp.concatenate([dst, pad_idx]).reshape(NW, NCHUNKS, CHUNK)

  x_pad = jnp.zeros((N_PAD, IN_CH), x.dtype).at[:N_NODES].set(x)

  ones = jnp.ones((CHUNK, 8), jnp.float32)
  zrows16 = jnp.zeros((ROWS_PER_TILE, 8), jnp.float32)
  zrows_d = jnp.zeros((128, D), jnp.float32)

  degp = _deg_pass(dst3, ones, zrows16)

  ys1, dinv = _tc_scale_matmul(degp, x_pad, W1)
  p1 = _edge_pass(ys1, src3, dst3, zrows_d)

  ys2a, ys2b = _tc_layer1_finish(p1, ys1, dinv, b1, W2)
  p2a = _edge_pass(ys2a, src3, dst3, zrows_d)
  p2b = _edge_pass(ys2b, src3, dst3, zrows_d)

  out = _tc_layer2_finish(p2a, p2b, ys2a, ys2b, dinv, b2)
  return out[:N_NODES]


def kernel(x, edge_index, W1, b1, W2, b2):
  return _gcn_block(x, edge_index, W1, b1, W2, b2)


# speedup vs baseline: 33.8871x; 1.0346x over previous
import jax
import jax.numpy as jnp
from jax import lax
from jax.experimental import pallas as pl
from jax.experimental.pallas import tpu as pltpu
from jax.experimental.pallas import tpu_sc as plsc

N_NODES = 10000
N_EDGES = 320000
IN_CH = 128
HID = 64
OUT_CH = 128

NC = 2
NS = 16
NW = NC * NS

CHUNK = 128
EPT = 10240
E_PAD = EPT * NW
NCHUNKS = E_PAD // CHUNK
CPT1 = NCHUNKS // NW
CPT2 = NCHUNKS // NS
N_PAD = 10112
ROWS_PER_TILE = N_PAD // NS
D = 64
NBUF = 4


def _sc_mesh():
  return plsc.VectorSubcoreMesh(core_axis_name="c", subcore_axis_name="s")


def _zero_table_and_preload(zrows_hbm, zrows_v, table_sh, s, psem, copies):
  del psem
  pltpu.sync_copy(zrows_hbm, zrows_v)
  row0 = s * ROWS_PER_TILE
  for z0 in (0, 128, 256, 384):
    pltpu.sync_copy(zrows_v, table_sh.at[pl.ds(row0 + z0, 128)])
  pltpu.sync_copy(zrows_v.at[pl.ds(0, ROWS_PER_TILE - 512)],
                  table_sh.at[pl.ds(row0 + 512, ROWS_PER_TILE - 512)])
  for src, dst in copies:
    pltpu.sync_copy(src, dst)
  return row0


def _deg_body(dstc_hbm, ones_hbm, zrows_hbm, out_hbm,
              idx_v, ones_v, zrows_v, table_sh, psem, ssem, osem):
  c = lax.axis_index("c")
  s = lax.axis_index("s")
  wid = s * NC + c

  row0 = _zero_table_and_preload(
      zrows_hbm, zrows_v, table_sh, s, psem,
      [(ones_hbm, ones_v),
       (dstc_hbm.at[pl.ds(wid * CPT1, CPT1)], idx_v)])
  plsc.subcore_barrier()

  @pl.loop(0, CPT1)
  def _fire(k):
    pltpu.async_copy(ones_v, table_sh.at[idx_v.at[k]], ssem, add=True).wait()

  plsc.subcore_barrier()

  pltpu.async_copy(
      table_sh.at[pl.ds(row0, ROWS_PER_TILE)],
      out_hbm.at[c, pl.ds(row0, ROWS_PER_TILE)],
      osem,
  ).wait()


def _deg_pass(dstc, ones, zrows):
  kfn = pl.kernel(
      _deg_body,
      out_type=jax.ShapeDtypeStruct((NC, N_PAD, 8), jnp.float32),
      mesh=_sc_mesh(),
      scratch_types=[
          pltpu.VMEM((CPT1, CHUNK), jnp.int32),
          pltpu.VMEM((CHUNK, 8), jnp.float32),
          pltpu.VMEM((128, 8), jnp.float32),
          pltpu.VMEM_SHARED((N_PAD, 8), jnp.float32),
          pltpu.SemaphoreType.DMA,
          pltpu.SemaphoreType.DMA,
          pltpu.SemaphoreType.DMA,
      ],
      compiler_params=pltpu.CompilerParams(use_tc_tiling_on_sc=False),
  )
  return kfn(dstc, ones, zrows)


def _edge_loop(ys_ref, src_v, dst_v, rows_v, table_sh, gsems, ssems, cpt):
  def gather(k, b):
    pltpu.async_copy(ys_ref.at[src_v.at[k]], rows_v.at[b], gsems[b])

  def wait_gather(k, b):
    pltpu.make_async_copy(ys_ref.at[src_v.at[k]], rows_v.at[b],
                          gsems[b]).wait()

  def scatter(k, b):
    pltpu.async_copy(rows_v.at[b], table_sh.at[dst_v.at[k]], ssems[b],
                     add=True)

  def wait_scatter(k, b):
    pltpu.make_async_copy(rows_v.at[b], table_sh.at[dst_v.at[k]],
                          ssems[b]).wait()

  gather(0, 0)
  gather(1, 1)

  @pl.loop(0, cpt, step=NBUF)
  def _chunks(k0):
    for b in range(NBUF):
      k = k0 + b
      bp2 = (b + 2) % NBUF

      @pl.when(k >= 2)
      def _wait_prev_scatter():
        wait_scatter(k - 2, bp2)

      @pl.when(k + 2 < cpt)
      def _prefetch():
        gather(k + 2, bp2)

      wait_gather(k, b)
      scatter(k, b)

  wait_scatter(cpt - 2, (cpt - 2) % NBUF)
  wait_scatter(cpt - 1, (cpt - 1) % NBUF)


def _edge1_body(ys_hbm, srcc_hbm, dstc_hbm, zrows_hbm, out_hbm,
                src_v, dst_v, rows_v, zrows_v, table_sh,
                g0, g1, g2, g3, s0, s1, s2, s3, psem, osem):
  c = lax.axis_index("c")
  s = lax.axis_index("s")
  wid = s * NC + c

  row0 = _zero_table_and_preload(
      zrows_hbm, zrows_v, table_sh, s, psem,
      [(srcc_hbm.at[pl.ds(wid * CPT1, CPT1)], src_v),
       (dstc_hbm.at[pl.ds(wid * CPT1, CPT1)], dst_v)])
  plsc.subcore_barrier()

  _edge_loop(ys_hbm, src_v, dst_v, rows_v, table_sh,
             (g0, g1, g2, g3), (s0, s1, s2, s3), CPT1)
  plsc.subcore_barrier()

  pltpu.async_copy(
      table_sh.at[pl.ds(row0, ROWS_PER_TILE)],
      out_hbm.at[c, pl.ds(row0, ROWS_PER_TILE)],
      osem,
  ).wait()


def _edge_pass1(ys_pad, srcc, dstc, zrows):
  kfn = pl.kernel(
      _edge1_body,
      out_type=jax.ShapeDtypeStruct((NC, N_PAD, D), jnp.float32),
      mesh=_sc_mesh(),
      scratch_types=[
          pltpu.VMEM((CPT1, CHUNK), jnp.int32),
          pltpu.VMEM((CPT1, CHUNK), jnp.int32),
          pltpu.VMEM((NBUF, CHUNK, D), jnp.float32),
          pltpu.VMEM((128, D), jnp.float32),
          pltpu.VMEM_SHARED((N_PAD, D), jnp.float32),
      ] + [pltpu.SemaphoreType.DMA] * 10,
      compiler_params=pltpu.CompilerParams(use_tc_tiling_on_sc=False),
  )
  return kfn(ys_pad, srcc, dstc, zrows)


def _edge2_body(ys3_hbm, srcc_hbm, dstc_hbm, zrows_hbm, out_hbm,
                src_v, dst_v, rows_v, zrows_v, table_sh,
                g0, g1, g2, g3, s0, s1, s2, s3, psem, osem):
  c = lax.axis_index("c")
  s = lax.axis_index("s")

  row0 = _zero_table_and_preload(
      zrows_hbm, zrows_v, table_sh, s, psem,
      [(srcc_hbm.at[pl.ds(s * CPT2, CPT2)], src_v),
       (dstc_hbm.at[pl.ds(s * CPT2, CPT2)], dst_v)])
  plsc.subcore_barrier()

  _edge_loop(ys3_hbm.at[c], src_v, dst_v, rows_v, table_sh,
             (g0, g1, g2, g3), (s0, s1, s2, s3), CPT2)
  plsc.subcore_barrier()

  pltpu.async_copy(
      table_sh.at[pl.ds(row0, ROWS_PER_TILE)],
      out_hbm.at[c, pl.ds(row0, ROWS_PER_TILE)],
      osem,
  ).wait()


def _edge_pass2(ys3, srcc, dstc, zrows):
  kfn = pl.kernel(
      _edge2_body,
      out_type=jax.ShapeDtypeStruct((NC, N_PAD, D), jnp.float32),
      mesh=_sc_mesh(),
      scratch_types=[
          pltpu.VMEM((CPT2, CHUNK), jnp.int32),
          pltpu.VMEM((CPT2, CHUNK), jnp.int32),
          pltpu.VMEM((NBUF, CHUNK, D), jnp.float32),
          pltpu.VMEM((128, D), jnp.float32),
          pltpu.VMEM_SHARED((N_PAD, D), jnp.float32),
      ] + [pltpu.SemaphoreType.DMA] * 10,
      compiler_params=pltpu.CompilerParams(use_tc_tiling_on_sc=False),
  )
  return kfn(ys3, srcc, dstc, zrows)


def _tc_scale_matmul_body(degp_ref, x_ref, w_ref, ys_ref, dinv_ref):
  deg = degp_ref[0] + degp_ref[1] + 1.0
  dinv = lax.rsqrt(deg)
  dinv_ref[...] = dinv
  xw = jnp.dot(x_ref[...], w_ref[...], preferred_element_type=jnp.float32)
  ys_ref[...] = xw * dinv


def _tc_scale_matmul(degp, x_pad, w):
  d_out = w.shape[1]
  deg_cols = degp[:, :, 0:1]
  return pl.pallas_call(
      _tc_scale_matmul_body,
      out_shape=(
          jax.ShapeDtypeStruct((N_PAD, d_out), jnp.float32),
          jax.ShapeDtypeStruct((N_PAD, 1), jnp.float32),
      ),
  )(deg_cols, x_pad, w)


def _tc_layer1_finish_body(p_ref, ys_ref, dinv_ref, b_ref, w2_ref, ys3_ref):
  agg = p_ref[0] + p_ref[1] + ys_ref[...]
  h = jnp.maximum(agg * dinv_ref[...] + b_ref[...], 0.0)
  hw = jnp.dot(h, w2_ref[...], preferred_element_type=jnp.float32)
  ys2 = hw * dinv_ref[...]
  ys3_ref[0] = ys2[:, :D]
  ys3_ref[1] = ys2[:, D:]


def _tc_layer1_finish(partials, ys1, dinv, b1, w2):
  return pl.pallas_call(
      _tc_layer1_finish_body,
      out_shape=jax.ShapeDtypeStruct((NC, N_PAD, D), jnp.float32),
  )(partials, ys1, dinv, b1.reshape(1, HID), w2)


def _tc_layer2_finish_body(p_ref, ys3_ref, dinv_ref, b_ref, out_ref):
  agga = p_ref[0] + ys3_ref[0]
  aggb = p_ref[1] + ys3_ref[1]
  out_ref[:, :D] = agga * dinv_ref[...] + b_ref[:, :D]
  out_ref[:, D:] = aggb * dinv_ref[...] + b_ref[:, D:]


def _tc_layer2_finish(p2, ys3, dinv, b2):
  return pl.pallas_call(
      _tc_layer2_finish_body,
      out_shape=jax.ShapeDtypeStruct((N_PAD, OUT_CH), jnp.float32),
  )(p2, ys3, dinv, b2.reshape(1, OUT_CH))


def _gcn_block(x, edge_index, W1, b1, W2, b2):
  src = edge_index[0].astype(jnp.int32)
  dst = edge_index[1].astype(jnp.int32)
  npad_e = E_PAD - N_EDGES
  pad_idx = N_NODES + (jnp.arange(npad_e, dtype=jnp.int32) % (N_PAD - N_NODES))
  srcc = jnp.concatenate([src, pad_idx]).reshape(NCHUNKS, CHUNK)
  dstc = jnp.concatenate([dst, pad_idx]).reshape(NCHUNKS, CHUNK)

  x_pad = jnp.zeros((N_PAD, IN_CH), x.dtype).at[:N_NODES].set(x)

  ones = jnp.ones((CHUNK, 8), jnp.float32)
  zrows8 = jnp.zeros((128, 8), jnp.float32)
  zrows_d = jnp.zeros((128, D), jnp.float32)

  degp = _deg_pass(dstc, ones, zrows8)

  ys1, dinv = _tc_scale_matmul(degp, x_pad, W1)
  p1 = _edge_pass1(ys1, srcc, dstc, zrows_d)

  ys3 = _tc_layer1_finish(p1, ys1, dinv, b1, W2)
  p2 = _edge_pass2(ys3, srcc, dstc, zrows_d)

  out = _tc_layer2_finish(p2, ys3, dinv, b2)
  return out[:N_NODES]


def kernel(x, edge_index, W1, b1, W2, b2):
  return _gcn_block(x, edge_index, W1, b1, W2, b2)
